# baseline (speedup 1.0000x reference)
import functools

import jax
import jax.numpy as jnp
from jax import lax
from jax.experimental import pallas as pl
from jax.experimental.pallas import tpu as pltpu
from jax.experimental.pallas import tpu_sc as plsc

NC = 2
NS = 16
LANES = 16

CHUNK = 128


def _sc_mesh():
    return plsc.VectorSubcoreMesh(
        core_axis_name="c", subcore_axis_name="s", num_cores=NC, num_subcores=NS
    )


def _degree_kernel(n_nodes, n_chunks):
    acc_len = n_nodes + 8

    @functools.partial(
        pl.kernel,
        out_type=jax.ShapeDtypeStruct((NC * n_nodes,), jnp.float32),
        mesh=_sc_mesh(),
        scratch_types=[
            pltpu.VMEM((n_chunks, CHUNK), jnp.int32),
            pltpu.VMEM((CHUNK,), jnp.float32),
            pltpu.VMEM((1024,), jnp.float32),
            pltpu.VMEM_SHARED((acc_len,), jnp.float32),
            pltpu.SemaphoreType.DMA,
        ],
    )
    def deg_kernel(dst3_hbm, out_hbm, didx, ones_v, zbuf, acc_sh, sem):
        c = lax.axis_index("c")
        s = lax.axis_index("s")
        tile = c * NS + s

        pltpu.sync_copy(dst3_hbm.at[tile], didx)

        def fill_ones(i, _):
            ones_v[pl.ds(i * LANES, LANES)] = jnp.ones((LANES,), jnp.float32)
            return 0
        lax.fori_loop(0, CHUNK // LANES, fill_ones, 0)

        def fill_zero(i, _):
            zbuf[pl.ds(i * LANES, LANES)] = jnp.zeros((LANES,), jnp.float32)
            return 0
        lax.fori_loop(0, 1024 // LANES, fill_zero, 0)

        @pl.when(s == 0)
        def _():
            n_z = n_nodes // 1024

            def zero_acc(i, _):
                pltpu.sync_copy(zbuf, acc_sh.at[pl.ds(i * 1024, 1024)])
                return 0
            lax.fori_loop(0, n_z, zero_acc, 0)
            rem = n_nodes - n_z * 1024
            if rem:
                pltpu.sync_copy(
                    zbuf.at[pl.ds(0, rem)], acc_sh.at[pl.ds(n_z * 1024, rem)]
                )

        plsc.subcore_barrier()

        def fire(j, _):
            pltpu.async_copy(ones_v, acc_sh.at[didx.at[j]], sem, add=True)
            return 0
        lax.fori_loop(0, n_chunks, fire, 0)

        def drain(j, _):
            pltpu.make_async_copy(ones_v, acc_sh.at[didx.at[0]], sem).wait()
            return 0
        lax.fori_loop(0, n_chunks, drain, 0)

        plsc.subcore_barrier()

        n_oc = n_nodes // 1024
        oc_per_tile = (n_oc + NS - 1) // NS

        def copy_out(i, _):
            k = s + i * NS

            @pl.when(k < n_oc)
            def _():
                pltpu.sync_copy(acc_sh.at[pl.ds(k * 1024, 1024)], zbuf)
                pltpu.sync_copy(
                    zbuf, out_hbm.at[pl.ds(c * n_nodes + k * 1024, 1024)]
                )
            return 0
        lax.fori_loop(0, oc_per_tile, copy_out, 0)
        rem = n_nodes - n_oc * 1024
        if rem:
            @pl.when(s == NS - 1)
            def _():
                pltpu.sync_copy(
                    acc_sh.at[pl.ds(n_oc * 1024, rem)], zbuf.at[pl.ds(0, rem)]
                )
                pltpu.sync_copy(
                    zbuf.at[pl.ds(0, rem)],
                    out_hbm.at[pl.ds(c * n_nodes + n_oc * 1024, rem)],
                )

    return deg_kernel


def _scatter_kernel(n_nodes, n_chunks, d):
    assert n_chunks % 2 == 0
    n_iters = n_chunks // 2
    acc_rows = n_nodes + 8
    nz_full = acc_rows // CHUNK
    nz_tail = acc_rows - nz_full * CHUNK
    n_row_chunks = n_nodes // CHUNK
    row_tail = n_nodes - n_row_chunks * CHUNK
    chunks_per_tile = (nz_full + NS - 1) // NS

    @functools.partial(
        pl.kernel,
        out_type=jax.ShapeDtypeStruct((NC * n_nodes, d), jnp.float32),
        mesh=_sc_mesh(),
        scratch_types=[
            pltpu.VMEM((CHUNK,), jnp.int32),
            pltpu.VMEM((CHUNK,), jnp.int32),
            pltpu.VMEM((n_chunks, CHUNK), jnp.int32),
            pltpu.VMEM((CHUNK, d), jnp.float32),
            pltpu.VMEM((CHUNK, d), jnp.float32),
            pltpu.VMEM_SHARED((acc_rows, d), jnp.float32),
            pltpu.SemaphoreType.DMA,
            pltpu.SemaphoreType.DMA,
            pltpu.SemaphoreType.DMA,
        ],
    )
    def scat_kernel(src3_hbm, dst3_hbm, g_hbm, out_hbm,
                    sidx0, sidx1, didx, rows0, rows1, acc_sh,
                    isem, gsem, ssem):
        c = lax.axis_index("c")
        s = lax.axis_index("s")
        tile = c * NS + s

        pltpu.sync_copy(dst3_hbm.at[tile], didx)

        def fill_row(i, _):
            def fill_lane(j, _):
                rows0[i, pl.ds(j * LANES, LANES)] = jnp.zeros((LANES,), jnp.float32)
                return 0
            lax.fori_loop(0, d // LANES, fill_lane, 0)
            return 0
        lax.fori_loop(0, CHUNK, fill_row, 0)

        def zero_rows(i, _):
            k = s + i * NS

            @pl.when(k < nz_full)
            def _():
                pltpu.sync_copy(rows0, acc_sh.at[pl.ds(k * CHUNK, CHUNK)])
            return 0
        lax.fori_loop(0, chunks_per_tile, zero_rows, 0)
        if nz_tail:
            @pl.when(s == NS - 1)
            def _():
                pltpu.sync_copy(
                    rows0.at[pl.ds(0, nz_tail)],
                    acc_sh.at[pl.ds(nz_full * CHUNK, nz_tail)],
                )

        plsc.subcore_barrier()

        def load_sidx(j, slot):
            pltpu.async_copy(src3_hbm.at[tile, j], slot, isem)

        def wait_sidx(slot):
            pltpu.make_async_copy(src3_hbm.at[tile, 0], slot, isem).wait()

        def gather(slot, buf):
            pltpu.async_copy(g_hbm.at[slot], buf, gsem)

        def wait_gather(buf):
            pltpu.make_async_copy(g_hbm.at[sidx0], buf, gsem).wait()

        def scatter(j, buf):
            pltpu.async_copy(buf, acc_sh.at[didx.at[j]], ssem, add=True)

        def wait_scatter(buf):
            pltpu.make_async_copy(buf, acc_sh.at[didx.at[0]], ssem).wait()

        pltpu.sync_copy(src3_hbm.at[tile, 0], sidx0)
        gather(sidx0, rows0)
        pltpu.sync_copy(src3_hbm.at[tile, 1], sidx1)
        gather(sidx1, rows1)

        def body(i, _):
            j0 = 2 * i
            j1 = 2 * i + 1
            wait_gather(rows0)
            scatter(j0, rows0)

            @pl.when(j0 + 2 < n_chunks)
            def _():
                load_sidx(j0 + 2, sidx0)
            wait_gather(rows1)
            scatter(j1, rows1)

            @pl.when(j1 + 2 < n_chunks)
            def _():
                load_sidx(j1 + 2, sidx1)
            wait_scatter(rows0)

            @pl.when(j0 + 2 < n_chunks)
            def _():
                wait_sidx(sidx0)
                gather(sidx0, rows0)
            wait_scatter(rows1)

            @pl.when(j1 + 2 < n_chunks)
            def _():
                wait_sidx(sidx1)
                gather(sidx1, rows1)
            return 0
        lax.fori_loop(0, n_iters, body, 0)

        plsc.subcore_barrier()

        out_base = c * n_nodes

        def copy_out(i, _):
            k = s + i * NS

            @pl.when(k < n_row_chunks)
            def _():
                pltpu.sync_copy(
                    acc_sh.at[pl.ds(k * CHUNK, CHUNK)],
                    out_hbm.at[pl.ds(out_base + k * CHUNK, CHUNK)],
                )
            return 0
        lax.fori_loop(0, chunks_per_tile, copy_out, 0)
        if row_tail:
            @pl.when(s == 0)
            def _():
                pltpu.sync_copy(
                    acc_sh.at[pl.ds(n_row_chunks * CHUNK, row_tail)],
                    out_hbm.at[pl.ds(out_base + n_row_chunks * CHUNK, row_tail)],
                )

    return scat_kernel


def _gW_body(x_ref, w_ref, degp_ref, g_ref):
    n = x_ref.shape[0]
    h = jnp.dot(x_ref[...], w_ref[...], preferred_element_type=jnp.float32)
    deg = degp_ref[0:n] + degp_ref[n:2 * n] + 1.0
    dis = lax.rsqrt(deg)
    g_ref[...] = h * dis[:, None]


def _epilogue_body(s_ref, g_ref, degp_ref, b_ref, gamma_ref, beta_ref, y_ref):
    n = g_ref.shape[0]
    deg = degp_ref[0:n] + degp_ref[n:2 * n] + 1.0
    dis = lax.rsqrt(deg)
    total = s_ref[0:n, :] + s_ref[n:2 * n, :] + g_ref[...]
    out = total * dis[:, None] + b_ref[...][None, :]
    mean = jnp.mean(out, axis=0)
    var = jnp.mean((out - mean[None, :]) ** 2, axis=0)
    y = gamma_ref[...][None, :] * (out - mean[None, :]) * lax.rsqrt(
        var[None, :] + 1e-5
    ) + beta_ref[...][None, :]
    y_ref[...] = jnp.maximum(y, 0.0)


def kernel(x, edge_index, W, b, gamma, beta):
    n_nodes, d_in = x.shape
    d_out = W.shape[1]
    n_edges = edge_index.shape[1]
    src = edge_index[0]
    dst = edge_index[1]

    n_tiles = NC * NS
    n_chunks = -(-n_edges // (n_tiles * CHUNK))
    n_chunks += n_chunks % 2
    e_pad = n_tiles * n_chunks * CHUNK
    pad = e_pad - n_edges
    src_p = jnp.concatenate([src, jnp.zeros((pad,), jnp.int32)])
    dst_p = jnp.concatenate([dst, jnp.full((pad,), n_nodes, jnp.int32)])
    src3 = src_p.reshape(n_tiles, n_chunks, CHUNK)
    dst3 = dst_p.reshape(n_tiles, n_chunks, CHUNK)

    degp = _degree_kernel(n_nodes, n_chunks)(dst3)

    g = pl.pallas_call(
        _gW_body,
        out_shape=jax.ShapeDtypeStruct((n_nodes, d_out), jnp.float32),
    )(x, W, degp)

    s_partial = _scatter_kernel(n_nodes, n_chunks, d_out)(src3, dst3, g)

    y = pl.pallas_call(
        _epilogue_body,
        out_shape=jax.ShapeDtypeStruct((n_nodes, d_out), jnp.float32),
    )(s_partial, g, degp, b, gamma, beta)
    return y
